# baseline (speedup 1.0000x reference)
import functools

import jax
import jax.numpy as jnp
from jax import lax
from jax.experimental import pallas as pl
from jax.experimental.pallas import tpu as pltpu
from jax.experimental.pallas import tpu_sc as plsc

DIM = 6144
D3 = DIM // 3
B, S = 4, 2048
BS = 512

_SC_MESH = plsc.VectorSubcoreMesh(core_axis_name="c", subcore_axis_name="s")


@functools.partial(
    pl.kernel,
    out_type=jax.ShapeDtypeStruct((B, 1, DIM), jnp.float32),
    mesh=_SC_MESH,
    scratch_types=[
        pltpu.VMEM((B,), jnp.int32),
        pltpu.VMEM((B, D3), jnp.float32),
        pltpu.SemaphoreType.DMA,
    ],
)
def _gather_bias(q_hbm, k_hbm, v_hbm, idx_hbm, out_hbm, idx_v, rows_v, sem):
    wid = lax.axis_index("s") * 2 + lax.axis_index("c")
    for t, tab in enumerate((q_hbm, k_hbm, v_hbm)):
        @pl.when(wid == t)
        def _(tab=tab, t=t):
            pltpu.sync_copy(idx_hbm, idx_v)
            pltpu.async_copy(tab.at[idx_v], rows_v, sem).wait()
            pltpu.sync_copy(rows_v, out_hbm.at[:, 0, pl.ds(t * D3, D3)])


def _add_body(x_ref, bias_ref, o_ref):
    o_ref[0] = x_ref[0] + bias_ref[0]


def kernel(x, bias_idx, q_bias, k_bias, v_bias):
    idx = bias_idx.astype(jnp.int32)
    bias = _gather_bias(q_bias, k_bias, v_bias, idx)
    grid = (B, S // BS)
    return pl.pallas_call(
        _add_body,
        grid=grid,
        in_specs=[
            pl.BlockSpec((1, BS, DIM), lambda b, s: (b, s, 0)),
            pl.BlockSpec((1, 1, DIM), lambda b, s: (b, 0, 0)),
        ],
        out_specs=pl.BlockSpec((1, BS, DIM), lambda b, s: (b, s, 0)),
        out_shape=jax.ShapeDtypeStruct((B, S, DIM), jnp.float32),
        compiler_params=pltpu.CompilerParams(
            dimension_semantics=("arbitrary", "arbitrary"),
        ),
    )(x, bias)


# speedup vs baseline: 1.1452x; 1.1452x over previous
import jax
import jax.numpy as jnp
from jax.experimental import pallas as pl
from jax.experimental.pallas import tpu as pltpu

DIM = 6144
D3 = DIM // 3
B, S = 4, 2048
BS = 256


def _add_body(idx_ref, x_ref, q_hbm, k_hbm, v_hbm, o_ref,
              qs, ks, vs, sem):
    b = pl.program_id(0)
    s = pl.program_id(1)

    @pl.when(jnp.logical_and(b == 0, s == 0))
    def _fetch_bias():
        copies = []
        for bb in range(B):
            i = idx_ref[bb]
            for tab, dst in ((q_hbm, qs), (k_hbm, ks), (v_hbm, vs)):
                cp = pltpu.make_async_copy(
                    tab.at[pl.ds(i, 1), :], dst.at[pl.ds(bb, 1), :], sem)
                cp.start()
                copies.append(cp)
        for cp in copies:
            cp.wait()

    xr = x_ref[0]
    o_ref[0, :, 0 * D3:1 * D3] = xr[:, 0 * D3:1 * D3] + qs[pl.ds(b, 1), :]
    o_ref[0, :, 1 * D3:2 * D3] = xr[:, 1 * D3:2 * D3] + ks[pl.ds(b, 1), :]
    o_ref[0, :, 2 * D3:3 * D3] = xr[:, 2 * D3:3 * D3] + vs[pl.ds(b, 1), :]


def kernel(x, bias_idx, q_bias, k_bias, v_bias):
    idx = bias_idx.astype(jnp.int32)
    grid = (B, S // BS)
    return pl.pallas_call(
        _add_body,
        grid_spec=pltpu.PrefetchScalarGridSpec(
            num_scalar_prefetch=1,
            grid=grid,
            in_specs=[
                pl.BlockSpec((1, BS, DIM), lambda b, s, i: (b, s, 0)),
                pl.BlockSpec(memory_space=pl.ANY),
                pl.BlockSpec(memory_space=pl.ANY),
                pl.BlockSpec(memory_space=pl.ANY),
            ],
            out_specs=pl.BlockSpec((1, BS, DIM), lambda b, s, i: (b, s, 0)),
            scratch_shapes=[
                pltpu.VMEM((B, D3), jnp.float32),
                pltpu.VMEM((B, D3), jnp.float32),
                pltpu.VMEM((B, D3), jnp.float32),
                pltpu.SemaphoreType.DMA,
            ],
        ),
        out_shape=jax.ShapeDtypeStruct((B, S, DIM), jnp.float32),
        compiler_params=pltpu.CompilerParams(
            dimension_semantics=("arbitrary", "arbitrary"),
        ),
    )(idx, x, q_bias, k_bias, v_bias)


# speedup vs baseline: 1.1531x; 1.0069x over previous
import jax
import jax.numpy as jnp
from jax.experimental import pallas as pl
from jax.experimental.pallas import tpu as pltpu

DIM = 6144
D3 = DIM // 3
B, S = 4, 2048
BS = 512


def _add_body(idx_ref, x_ref, q_hbm, k_hbm, v_hbm, o_ref,
              qs, ks, vs, sem):
    b = pl.program_id(0)
    s = pl.program_id(1)

    @pl.when(jnp.logical_and(b == 0, s == 0))
    def _fetch_bias():
        copies = []
        for bb in range(B):
            i = idx_ref[bb]
            for tab, dst in ((q_hbm, qs), (k_hbm, ks), (v_hbm, vs)):
                cp = pltpu.make_async_copy(
                    tab.at[pl.ds(i, 1), :], dst.at[pl.ds(bb, 1), :], sem)
                cp.start()
                copies.append(cp)
        for cp in copies:
            cp.wait()

    xr = x_ref[0]
    o_ref[0, :, 0 * D3:1 * D3] = xr[:, 0 * D3:1 * D3] + qs[pl.ds(b, 1), :]
    o_ref[0, :, 1 * D3:2 * D3] = xr[:, 1 * D3:2 * D3] + ks[pl.ds(b, 1), :]
    o_ref[0, :, 2 * D3:3 * D3] = xr[:, 2 * D3:3 * D3] + vs[pl.ds(b, 1), :]


def kernel(x, bias_idx, q_bias, k_bias, v_bias):
    idx = bias_idx.astype(jnp.int32)
    grid = (B, S // BS)
    return pl.pallas_call(
        _add_body,
        grid_spec=pltpu.PrefetchScalarGridSpec(
            num_scalar_prefetch=1,
            grid=grid,
            in_specs=[
                pl.BlockSpec((1, BS, DIM), lambda b, s, i: (b, s, 0)),
                pl.BlockSpec(memory_space=pl.ANY),
                pl.BlockSpec(memory_space=pl.ANY),
                pl.BlockSpec(memory_space=pl.ANY),
            ],
            out_specs=pl.BlockSpec((1, BS, DIM), lambda b, s, i: (b, s, 0)),
            scratch_shapes=[
                pltpu.VMEM((B, D3), jnp.float32),
                pltpu.VMEM((B, D3), jnp.float32),
                pltpu.VMEM((B, D3), jnp.float32),
                pltpu.SemaphoreType.DMA,
            ],
        ),
        out_shape=jax.ShapeDtypeStruct((B, S, DIM), jnp.float32),
        compiler_params=pltpu.CompilerParams(
            dimension_semantics=("parallel", "arbitrary"),

        ),
    )(idx, x, q_bias, k_bias, v_bias)
